# Initial kernel scaffold; baseline (speedup 1.0000x reference)
import jax
import jax.numpy as jnp
from jax.experimental import pallas as pl


def kernel(token_seqs_embeddings, token_type_sequences, sequences_lengths, W_ih, W_hh, b_ih, b_hh, ln_gamma, ln_beta):
    raise NotImplementedError("write your pallas kernel here")


# speedup vs baseline: 6.3018x; 6.3018x over previous
import jax
import jax.numpy as jnp
from jax.experimental import pallas as pl
from jax.experimental.pallas import tpu as pltpu

_B, _S, _D = 16, 2048, 128
_H = _D
_CHUNK = 256
_IGNORE_KINDS = (0, 1)


def _gru_ln_kernel(keep_ref, xT_ref, wih_ref, whh_ref, bih_ref, bhh_ref,
                   g_ref, beta_ref, outT_ref, h_ref):
    @pl.when(pl.program_id(0) == 0)
    def _init():
        h_ref[...] = jnp.zeros_like(h_ref)

    wih = wih_ref[...]
    whh = whh_ref[...]
    bih = bih_ref[...]
    bhh = bhh_ref[...]
    gamma = g_ref[...]
    beta = beta_ref[...]

    def step(t, h):
        x_t = xT_ref[pl.ds(t, 1), :, :][0]
        gx = jnp.dot(x_t, wih, preferred_element_type=jnp.float32) + bih
        gh = jnp.dot(h, whh, preferred_element_type=jnp.float32) + bhh
        r = jax.nn.sigmoid(gx[:, 0:_H] + gh[:, 0:_H])
        z = jax.nn.sigmoid(gx[:, _H:2 * _H] + gh[:, _H:2 * _H])
        n = jnp.tanh(gx[:, 2 * _H:] + r * gh[:, 2 * _H:])
        h_new = (1.0 - z) * n + z * h
        k = keep_ref[pl.ds(t, 1), :, :][0]
        h2 = jnp.where(k > 0.0, h_new, h)
        mu = jnp.mean(h_new, axis=-1, keepdims=True)
        var = jnp.mean((h_new - mu) ** 2, axis=-1, keepdims=True)
        ln = (h_new - mu) * jax.lax.rsqrt(var + 1e-5) * gamma + beta
        outT_ref[pl.ds(t, 1), :, :] = jnp.where(k > 0.0, ln, x_t)[None]
        return h2

    h_ref[...] = jax.lax.fori_loop(0, _CHUNK, step, h_ref[...])


def kernel(token_seqs_embeddings, token_type_sequences, sequences_lengths,
           W_ih, W_hh, b_ih, b_hh, ln_gamma, ln_beta):
    del sequences_lengths
    x = token_seqs_embeddings
    b, s, d = x.shape

    keep = jnp.ones((b, s), dtype=bool)
    for kind in _IGNORE_KINDS:
        keep = jnp.logical_and(keep, token_type_sequences != kind)
    keepf = jnp.swapaxes(keep.astype(jnp.float32), 0, 1)[:, :, None]

    xT = jnp.swapaxes(x, 0, 1)

    grid = (s // _CHUNK,)
    outT = pl.pallas_call(
        _gru_ln_kernel,
        grid=grid,
        in_specs=[
            pl.BlockSpec((_CHUNK, b, 1), lambda i: (i, 0, 0)),
            pl.BlockSpec((_CHUNK, b, d), lambda i: (i, 0, 0)),
            pl.BlockSpec((d, 3 * _H), lambda i: (0, 0)),
            pl.BlockSpec((d, 3 * _H), lambda i: (0, 0)),
            pl.BlockSpec((1, 3 * _H), lambda i: (0, 0)),
            pl.BlockSpec((1, 3 * _H), lambda i: (0, 0)),
            pl.BlockSpec((1, d), lambda i: (0, 0)),
            pl.BlockSpec((1, d), lambda i: (0, 0)),
        ],
        out_specs=pl.BlockSpec((_CHUNK, b, d), lambda i: (i, 0, 0)),
        out_shape=jax.ShapeDtypeStruct((s, b, d), x.dtype),
        scratch_shapes=[pltpu.VMEM((b, d), jnp.float32)],
        compiler_params=pltpu.CompilerParams(
            dimension_semantics=("arbitrary",),
        ),
    )(keepf, xT, W_ih, W_hh, b_ih.reshape(1, -1), b_hh.reshape(1, -1),
      ln_gamma.reshape(1, -1), ln_beta.reshape(1, -1))

    return jnp.swapaxes(outT, 0, 1)


# speedup vs baseline: 13.0902x; 2.0772x over previous
import jax
import jax.numpy as jnp
from jax.experimental import pallas as pl
from jax.experimental.pallas import tpu as pltpu

_B, _S, _D = 16, 2048, 128
_H = _D
_CHUNK = 256
_IGNORE_KINDS = (0, 1)
_ZBIG = 1e9


def _gru_ln_kernel(keep_ref, xT_ref, wih_ref, whh_ref, brz_ref, bhhn_ref,
                   g_ref, beta_ref, outT_ref, gx_ref, hall_ref, h_ref):
    @pl.when(pl.program_id(0) == 0)
    def _init():
        h_ref[...] = jnp.zeros_like(h_ref)

    n_rows = _CHUNK * _B
    x2d = xT_ref[...].reshape(n_rows, _D)
    keep2d = keep_ref[...].reshape(n_rows, 1)

    gx = jnp.dot(x2d, wih_ref[...], preferred_element_type=jnp.float32)
    gx = gx + brz_ref[...]
    gx_ref[:, 0:_H] = gx[:, 0:_H]
    gx_ref[:, _H:2 * _H] = gx[:, _H:2 * _H] + _ZBIG * (1.0 - keep2d)
    gx_ref[:, 2 * _H:] = gx[:, 2 * _H:]

    whh = whh_ref[...]
    bhhn = bhhn_ref[...]

    def step(t, h):
        gx_t = gx_ref[pl.ds(t * _B, _B), :]
        gh = jnp.dot(h, whh, preferred_element_type=jnp.float32)
        r = jax.nn.sigmoid(gx_t[:, 0:_H] + gh[:, 0:_H])
        z = jax.nn.sigmoid(gx_t[:, _H:2 * _H] + gh[:, _H:2 * _H])
        n = jnp.tanh(gx_t[:, 2 * _H:] + r * (gh[:, 2 * _H:] + bhhn))
        h = (1.0 - z) * n + z * h
        hall_ref[pl.ds(t * _B, _B), :] = h
        return h

    h_ref[...] = jax.lax.fori_loop(0, _CHUNK, step, h_ref[...], unroll=8)

    hall = hall_ref[...]
    mu = jnp.mean(hall, axis=-1, keepdims=True)
    var = jnp.mean((hall - mu) ** 2, axis=-1, keepdims=True)
    ln = (hall - mu) * jax.lax.rsqrt(var + 1e-5) * g_ref[...] + beta_ref[...]
    res = jnp.where(keep2d > 0.0, ln, x2d)
    outT_ref[...] = res.reshape(_CHUNK, _B, _D)


def kernel(token_seqs_embeddings, token_type_sequences, sequences_lengths,
           W_ih, W_hh, b_ih, b_hh, ln_gamma, ln_beta):
    del sequences_lengths
    x = token_seqs_embeddings
    b, s, d = x.shape

    keep = jnp.ones((b, s), dtype=bool)
    for kind in _IGNORE_KINDS:
        keep = jnp.logical_and(keep, token_type_sequences != kind)
    keepf = jnp.swapaxes(keep.astype(jnp.float32), 0, 1)[:, :, None]

    xT = jnp.swapaxes(x, 0, 1)

    brz = b_ih + jnp.concatenate(
        [b_hh[0:_H], b_hh[_H:2 * _H], jnp.zeros((_H,), b_hh.dtype)])

    grid = (s // _CHUNK,)
    outT = pl.pallas_call(
        _gru_ln_kernel,
        grid=grid,
        in_specs=[
            pl.BlockSpec((_CHUNK, b, 1), lambda i: (i, 0, 0)),
            pl.BlockSpec((_CHUNK, b, d), lambda i: (i, 0, 0)),
            pl.BlockSpec((d, 3 * _H), lambda i: (0, 0)),
            pl.BlockSpec((d, 3 * _H), lambda i: (0, 0)),
            pl.BlockSpec((1, 3 * _H), lambda i: (0, 0)),
            pl.BlockSpec((1, _H), lambda i: (0, 0)),
            pl.BlockSpec((1, d), lambda i: (0, 0)),
            pl.BlockSpec((1, d), lambda i: (0, 0)),
        ],
        out_specs=pl.BlockSpec((_CHUNK, b, d), lambda i: (i, 0, 0)),
        out_shape=jax.ShapeDtypeStruct((s, b, d), x.dtype),
        scratch_shapes=[
            pltpu.VMEM((_CHUNK * b, 3 * _H), jnp.float32),
            pltpu.VMEM((_CHUNK * b, d), jnp.float32),
            pltpu.VMEM((b, d), jnp.float32),
        ],
        compiler_params=pltpu.CompilerParams(
            dimension_semantics=("arbitrary",),
        ),
    )(keepf, xT, W_ih, W_hh, brz.reshape(1, -1),
      b_hh[2 * _H:].reshape(1, -1), ln_gamma.reshape(1, -1),
      ln_beta.reshape(1, -1))

    return jnp.swapaxes(outT, 0, 1)
